# baseline (speedup 1.0000x reference)
import numpy as np
import jax
import jax.numpy as jnp
from jax import lax
from jax.experimental import pallas as pl
from jax.experimental.pallas import tpu as pltpu
from jax.experimental.pallas import tpu_sc as plsc

H = 524288
D = 3
F = 2
L = 16
B = 131072
NC, NS = 2, 16
NW = NC * NS
PT = B // NW
SB = 512
NSB = PT // SB
NVEC = SB // 16
NI = SB * 8
SEG = H // NS
OUTW = L * F
MASK = H - 1

_b = np.exp((np.log(512) - np.log(16)) / (L - 1))
RES = [float(np.floor(16 * _b ** i)) for i in range(L)]
_P64 = np.array([1, 2654435761, 805459861], dtype=np.int64)
PRIMES = [int(v) for v in _P64.astype(np.uint32).astype(np.int32)]


def _body(xf, tbl, out, x_v, idx0_v, idx1_v, w0_v, w1_v, rows0_v, rows1_v,
          out_v, spm0, spm1, sem, ssem):
    idxs = (idx0_v, idx1_v)
    ws = (w0_v, w1_v)
    rows = (rows0_v, rows1_v)
    cid = lax.axis_index("c")
    sid = lax.axis_index("s")
    wid = sid * jnp.int32(NC) + cid
    base = wid * jnp.int32(PT)
    sseg = sid * jnp.int32(SEG)

    iota = lax.iota(jnp.int32, 16)
    iota3 = iota * jnp.int32(D)
    iota32 = iota * jnp.int32(OUTW)
    one16 = jnp.full((16,), 1, jnp.int32)

    pltpu.sync_copy(xf.at[pl.ds(base * jnp.int32(D), PT * D)],
                    out_v.at[pl.ds(0, PT * D)])

    def x_t(v, c2):
        p0 = v * jnp.int32(16 * D)
        for d in range(D):
            vals = plsc.load_gather(out_v, [iota3 + (p0 + jnp.int32(d))])
            x_v[d, pl.ds(v * jnp.int32(16), 16)] = vals
        return c2

    lax.fori_loop(jnp.int32(0), jnp.int32(PT // 16), x_t, jnp.int32(0))

    spms = (spm0, spm1)

    def stage(lvl, buf):
        return pltpu.async_copy(
            tbl.at[pl.ds(jnp.int32(lvl * H) + sseg, SEG)],
            buf.at[pl.ds(sseg, SEG)], ssem)

    stage(0, spm0).wait()
    plsc.subcore_barrier()

    def sb_body(s, carry):
        scol = s * jnp.int32(SB)

        def make_pass_a(l, par):
            res = jnp.float32(RES[l])
            idx_b, w_b = idxs[par], ws[par]

            def pass_a(v, c2):
                colv = scol + v * jnp.int32(16)
                xs = [x_v[d, pl.ds(colv, 16)] for d in range(D)]
                scaled = [xs[d] * res for d in range(D)]
                gi = [t.astype(jnp.int32) for t in scaled]
                gf = [t.astype(jnp.float32) for t in gi]
                fr = [scaled[d] - gf[d] for d in range(D)]
                om = [1.0 - fr[d] for d in range(D)]
                a = [gi[d] * jnp.int32(PRIMES[d]) for d in range(D)]
                bb = [a[d] + jnp.int32(PRIMES[d]) for d in range(D)]
                wyz = [om[1] * om[2], fr[1] * om[2], om[1] * fr[2], fr[1] * fr[2]]
                q0 = v * jnp.int32(128)
                for c in range(8):
                    t = ((bb[0] if c & 1 else a[0])
                         ^ (bb[1] if c & 2 else a[1])
                         ^ (bb[2] if c & 4 else a[2]))
                    idx_b[pl.ds(q0 + jnp.int32(c * 16), 16)] = t & jnp.int32(MASK)
                    wc = (fr[0] if c & 1 else om[0]) * wyz[c >> 1]
                    w_b[pl.ds(q0 + jnp.int32(c * 16), 16)] = wc
                return c2

            lax.fori_loop(jnp.int32(0), jnp.int32(NVEC), pass_a, jnp.int32(0))

        make_pass_a(0, 0)
        for l in range(L):
            par = l & 1
            nxt = (l + 1) % L
            sdesc = stage(nxt, spms[nxt & 1])
            gd = pltpu.async_copy(spms[par].at[idxs[par]], rows[par], sem)
            if l + 1 < L:
                make_pass_a(l + 1, 1 - par)
            gd.wait()

            def pass_b(v, c2, _l=l, _w=ws[par], _r=rows[par]):
                acc0 = jnp.zeros((16,), jnp.float32)
                acc1 = jnp.zeros((16,), jnp.float32)
                q0 = v * jnp.int32(128)
                for c in range(8):
                    wc = _w[pl.ds(q0 + jnp.int32(c * 16), 16)]
                    r = _r[pl.ds(q0 + jnp.int32(c * 16), 16)]
                    f0 = plsc.bitcast(r << jnp.int32(16), jnp.float32)
                    f1 = plsc.bitcast(r & jnp.int32(-65536), jnp.float32)
                    acc0 = acc0 + wc * f0
                    acc1 = acc1 + wc * f1
                pos = v * jnp.int32(16 * OUTW) + iota32 + jnp.int32(2 * _l)
                plsc.store_scatter(out_v, [pos], acc0)
                plsc.store_scatter(out_v, [pos + one16], acc1)
                return c2

            lax.fori_loop(jnp.int32(0), jnp.int32(NVEC), pass_b, jnp.int32(0))
            sdesc.wait()
            plsc.subcore_barrier()
        pltpu.sync_copy(
            out_v, out.at[pl.ds((base + scol) * jnp.int32(OUTW), SB * OUTW)])
        return carry

    lax.fori_loop(jnp.int32(0), jnp.int32(NSB), sb_body, jnp.int32(0))


def _make():
    mesh = plsc.VectorSubcoreMesh(core_axis_name="c", subcore_axis_name="s")
    return pl.kernel(
        _body,
        out_type=jax.ShapeDtypeStruct((B * OUTW,), jnp.float32),
        mesh=mesh,
        compiler_params=pltpu.CompilerParams(needs_layout_passes=False),
        scratch_types=[
            pltpu.VMEM((D, PT), jnp.float32),
            pltpu.VMEM((NI,), jnp.int32),
            pltpu.VMEM((NI,), jnp.int32),
            pltpu.VMEM((NI,), jnp.float32),
            pltpu.VMEM((NI,), jnp.float32),
            pltpu.VMEM((NI,), jnp.int32),
            pltpu.VMEM((NI,), jnp.int32),
            pltpu.VMEM((SB * OUTW,), jnp.float32),
            pltpu.VMEM_SHARED((H,), jnp.int32),
            pltpu.VMEM_SHARED((H,), jnp.int32),
            pltpu.SemaphoreType.DMA,
            pltpu.SemaphoreType.DMA,
        ],
    )


def kernel(x, tables, resolutions, primes, border_adds):
    xf = x.astype(jnp.float32).reshape(B * D)
    packed = lax.bitcast_convert_type(
        tables.astype(jnp.bfloat16).reshape(L * H, F), jnp.int32)
    return _make()(xf, packed).reshape(B, OUTW)


# speedup vs baseline: 1.0344x; 1.0344x over previous
import numpy as np
import jax
import jax.numpy as jnp
from jax import lax
from jax.experimental import pallas as pl
from jax.experimental.pallas import tpu as pltpu
from jax.experimental.pallas import tpu_sc as plsc

H = 524288
D = 3
F = 2
L = 16
B = 131072
NC, NS = 2, 16
NW = NC * NS
PT = B // NW
SB = 1024
NSB = PT // SB
NVEC = SB // 16
NI = SB * 8
SEG = H // NS
OUTW = L * F
MASK = H - 1

_b = np.exp((np.log(512) - np.log(16)) / (L - 1))
RES = [float(np.floor(16 * _b ** i)) for i in range(L)]
_P64 = np.array([1, 2654435761, 805459861], dtype=np.int64)
PRIMES = [int(v) for v in _P64.astype(np.uint32).astype(np.int32)]


def _body(xf, tbl, out, x_v, idx_v, rows_v, out_v, spm0, spm1, sem, ssem):
    cid = lax.axis_index("c")
    sid = lax.axis_index("s")
    wid = sid * jnp.int32(NC) + cid
    base = wid * jnp.int32(PT)
    sseg = sid * jnp.int32(SEG)

    iota = lax.iota(jnp.int32, 16)
    iota3 = iota * jnp.int32(D)
    iota32 = iota * jnp.int32(OUTW)
    one16 = jnp.full((16,), 1, jnp.int32)

    pltpu.sync_copy(xf.at[pl.ds(base * jnp.int32(D), PT * D)],
                    out_v.at[pl.ds(0, PT * D)])

    def x_t(v, c2):
        p0 = v * jnp.int32(16 * D)
        for d in range(D):
            vals = plsc.load_gather(out_v, [iota3 + (p0 + jnp.int32(d))])
            x_v[d, pl.ds(v * jnp.int32(16), 16)] = vals
        return c2

    lax.fori_loop(jnp.int32(0), jnp.int32(PT // 16), x_t, jnp.int32(0))

    spms = (spm0, spm1)

    def stage(lvl, buf):
        return pltpu.async_copy(
            tbl.at[pl.ds(jnp.int32(lvl * H) + sseg, SEG)],
            buf.at[pl.ds(sseg, SEG)], ssem)

    stage(0, spm0).wait()
    plsc.subcore_barrier()

    def sb_body(s, carry):
        scol = s * jnp.int32(SB)
        for l in range(L):
            nxt = (l + 1) % L
            sdesc = stage(nxt, spms[nxt & 1])
            res = jnp.float32(RES[l])

            def pass_a(v, c2, _res=res):
                colv = scol + v * jnp.int32(16)
                xs = [x_v[d, pl.ds(colv, 16)] for d in range(D)]
                gi = [(xs[d] * _res).astype(jnp.int32) for d in range(D)]
                a = [gi[d] * jnp.int32(PRIMES[d]) for d in range(D)]
                bb = [a[d] + jnp.int32(PRIMES[d]) for d in range(D)]
                q0 = v * jnp.int32(128)
                for c in range(8):
                    t = ((bb[0] if c & 1 else a[0])
                         ^ (bb[1] if c & 2 else a[1])
                         ^ (bb[2] if c & 4 else a[2]))
                    idx_v[pl.ds(q0 + jnp.int32(c * 16), 16)] = t & jnp.int32(MASK)
                return c2

            lax.fori_loop(jnp.int32(0), jnp.int32(NVEC), pass_a, jnp.int32(0))
            pltpu.async_copy(spms[l & 1].at[idx_v], rows_v, sem).wait()

            def pass_b(v, c2, _l=l, _res=res):
                colv = scol + v * jnp.int32(16)
                xs = [x_v[d, pl.ds(colv, 16)] for d in range(D)]
                scaled = [xs[d] * _res for d in range(D)]
                gf = [t.astype(jnp.int32).astype(jnp.float32) for t in scaled]
                fr = [scaled[d] - gf[d] for d in range(D)]
                om = [1.0 - fr[d] for d in range(D)]
                wyz = [om[1] * om[2], fr[1] * om[2], om[1] * fr[2], fr[1] * fr[2]]
                acc0 = jnp.zeros((16,), jnp.float32)
                acc1 = jnp.zeros((16,), jnp.float32)
                q0 = v * jnp.int32(128)
                for c in range(8):
                    wc = (fr[0] if c & 1 else om[0]) * wyz[c >> 1]
                    r = rows_v[pl.ds(q0 + jnp.int32(c * 16), 16)]
                    f0 = plsc.bitcast(r << jnp.int32(16), jnp.float32)
                    f1 = plsc.bitcast(r & jnp.int32(-65536), jnp.float32)
                    acc0 = acc0 + wc * f0
                    acc1 = acc1 + wc * f1
                pos = v * jnp.int32(16 * OUTW) + iota32 + jnp.int32(2 * _l)
                plsc.store_scatter(out_v, [pos], acc0)
                plsc.store_scatter(out_v, [pos + one16], acc1)
                return c2

            lax.fori_loop(jnp.int32(0), jnp.int32(NVEC), pass_b, jnp.int32(0))
            sdesc.wait()
            plsc.subcore_barrier()
        pltpu.sync_copy(
            out_v, out.at[pl.ds((base + scol) * jnp.int32(OUTW), SB * OUTW)])
        return carry

    lax.fori_loop(jnp.int32(0), jnp.int32(NSB), sb_body, jnp.int32(0))


def _make():
    mesh = plsc.VectorSubcoreMesh(core_axis_name="c", subcore_axis_name="s")
    return pl.kernel(
        _body,
        out_type=jax.ShapeDtypeStruct((B * OUTW,), jnp.float32),
        mesh=mesh,
        compiler_params=pltpu.CompilerParams(needs_layout_passes=False),
        scratch_types=[
            pltpu.VMEM((D, PT), jnp.float32),
            pltpu.VMEM((NI,), jnp.int32),
            pltpu.VMEM((NI,), jnp.int32),
            pltpu.VMEM((SB * OUTW,), jnp.float32),
            pltpu.VMEM_SHARED((H,), jnp.int32),
            pltpu.VMEM_SHARED((H,), jnp.int32),
            pltpu.SemaphoreType.DMA,
            pltpu.SemaphoreType.DMA,
        ],
    )


def kernel(x, tables, resolutions, primes, border_adds):
    xf = x.astype(jnp.float32).reshape(B * D)
    packed = lax.bitcast_convert_type(
        tables.astype(jnp.bfloat16).reshape(L * H, F), jnp.int32)
    return _make()(xf, packed).reshape(B, OUTW)
